# baseline (speedup 1.0000x reference)
import functools

import jax
import jax.numpy as jnp
import numpy as np
from jax import lax
from jax.experimental import pallas as pl
from jax.experimental.pallas import tpu as pltpu
from jax.experimental.pallas import tpu_sc as plsc

T, D = 8192, 512
NUM_LAYERS = 3
NA = 8
NW = 32
T_SC = 1024
T_TC = T - T_SC
RPW = T_SC // NW
CH = min(64, RPW)
NCH = RPW // CH
NV = D // 16
TCB = 512
KDE_FACTOR = (D * 3.0 / 4.0) ** (-0.2)


def _noise_const():
    base_key = jax.random.key(1234)
    rows = [np.asarray(jax.random.normal(jax.random.fold_in(base_key, i),
                                         (D,), dtype=jnp.float32))
            for i in range(16)]
    order = list(range(0, 16, 2)) + list(range(1, 16, 2))
    return np.stack([rows[i] for i in order])


_NOISE = _noise_const()

@functools.lru_cache(maxsize=None)
def _get_sc_partial_sums():
    mesh = plsc.VectorSubcoreMesh(core_axis_name="c", subcore_axis_name="s")

    @functools.partial(
        pl.kernel,
        out_type=jax.ShapeDtypeStruct((NA, NW, D), jnp.float32),
        mesh=mesh,
        scratch_types=[
            pltpu.VMEM((CH, D), jnp.float32),
            pltpu.VMEM((CH, D), jnp.float32),
            pltpu.VMEM((NA, D), jnp.float32),
            pltpu.SemaphoreType.DMA,
            pltpu.SemaphoreType.DMA,
            pltpu.SemaphoreType.DMA,
        ],
    )
    def _sc_partial_sums(a0, a1, a2, a3, a4, a5, a6, a7, out,
                         buf0, buf1, accv, sem0, sem1, osem):
        wid = lax.axis_index("s") * 2 + lax.axis_index("c")
        base = wid * RPW
        refs = (a0, a1, a2, a3, a4, a5, a6, a7)
        bufs = (buf0, buf1)
        sems = (sem0, sem1)
        sched = [(a, c) for a in range(NA) for c in range(NCH)]

        def issue(k):
            a, c = sched[k]
            return pltpu.async_copy(
                refs[a].at[pl.ds(base + c * CH, CH)], bufs[k % 2],
                sems[k % 2])

        copies = [None, None]
        copies[0] = issue(0)
        out_copies = []
        accs = None
        for k, (a, c) in enumerate(sched):
            if k + 1 < len(sched):
                copies[(k + 1) % 2] = issue(k + 1)
            copies[k % 2].wait()
            buf = bufs[k % 2]
            if c == 0:
                accs = tuple(jnp.zeros((16,), jnp.float32)
                             for _ in range(NV))

            def row_body(r, acc, buf=buf):
                return tuple(acc[j] + buf[r, pl.ds(16 * j, 16)]
                             for j in range(NV))

            accs = plsc.parallel_loop(0, CH, 1, unroll=2,
                                      carry=accs)(row_body)
            if c == NCH - 1:
                for j in range(NV):
                    accv[a, pl.ds(16 * j, 16)] = accs[j]
                out_copies.append(
                    pltpu.async_copy(accv.at[a], out.at[a, wid], osem))
        for oc in out_copies:
            oc.wait()

    return _sc_partial_sums


def _tc_reduce_body(a0, a1, a2, a3, a4, a5, a6, a7, out_ref):
    @pl.when(pl.program_id(0) == 0)
    def _():
        out_ref[...] = jnp.zeros_like(out_ref)

    for i, ref in enumerate((a0, a1, a2, a3, a4, a5, a6, a7)):
        out_ref[i:i + 1, :] += jnp.sum(ref[...], axis=0, keepdims=True)


def _tc_reduce(arrs):
    grid = (T_TC // TCB,)
    in_spec = pl.BlockSpec((TCB, D), lambda i: (i + T_SC // TCB, 0))
    return pl.pallas_call(
        _tc_reduce_body,
        grid=grid,
        in_specs=[in_spec] * NA,
        out_specs=pl.BlockSpec((NA, D), lambda i: (0, 0)),
        out_shape=jax.ShapeDtypeStruct((NA, D), jnp.float32),
    )(*arrs)


def _split3(g):
    return g[:, :D], g[:, D:2 * D], g[:, 2 * D:]


def _tc_finish_body(parts_ref, tcsum_ref, w_ref, wih_ref, whh_ref, bih_ref,
                    bhh_ref, noise_ref, out_ref):
    sums = [jnp.sum(parts_ref[a * NW:(a + 1) * NW, :], axis=0, keepdims=True)
            for a in range(NA)]
    S = jnp.concatenate(sums, axis=0) + tcsum_ref[...]
    means = S * (1.0 / T)

    H0 = means[0:4]
    hp = means[4:8]
    bi = bih_ref[...]
    bh = bhh_ref[...]
    dn_t = (((1,), (1,)), ((), ()))

    for l in range(NUM_LAYERS):
        s_l = jnp.sum(hp, axis=0, keepdims=True)
        agg0 = lax.dot_general(s_l, w_ref[l], (((1,), (0,)), ((), ())),
                               preferred_element_type=jnp.float32)
        gi0 = lax.dot_general(agg0, wih_ref[...], dn_t,
                              preferred_element_type=jnp.float32) + bi
        gh0 = lax.dot_general(H0, whh_ref[...], dn_t,
                              preferred_element_type=jnp.float32) + bh
        ghp = lax.dot_general(hp, whh_ref[...], dn_t,
                              preferred_element_type=jnp.float32) + bh
        i_r, i_z, i_n = _split3(gi0)
        h_r, h_z, h_n = _split3(gh0)
        r = jax.nn.sigmoid(i_r + h_r)
        z = jax.nn.sigmoid(i_z + h_z)
        n = jnp.tanh(i_n + r * h_n)
        H0 = (1.0 - z) * n + z * H0
        i_rp, i_zp, i_np = _split3(bi)
        h_rp, h_zp, h_np = _split3(ghp)
        rp = jax.nn.sigmoid(i_rp + h_rp)
        zp = jax.nn.sigmoid(i_zp + h_zp)
        np_ = jnp.tanh(i_np + rp * h_np)
        hp = (1.0 - zp) * np_ + zp * hp

    c_fold = float(np.sqrt(np.log2(np.e) * 0.5))
    S1 = jnp.concatenate([H0, H0], axis=0) + 0.01 * noise_ref[0:8]
    S2 = jnp.concatenate([S[4:8], S[0:4]], axis=0) + 0.01 * noise_ref[8:16]
    A0 = jnp.concatenate([S1, S2], axis=0)
    msum = jnp.sum(A0, axis=1, keepdims=True) * (1.0 / D)
    var = jnp.sum((A0 - msum) ** 2, axis=1, keepdims=True) * (1.0 / (D - 1))
    a = A0 * (c_fold / (jnp.sqrt(var) * KDE_FACTOR))

    r_iota = lax.broadcasted_iota(jnp.int32, (16, 16 * D), 0)
    c_iota = lax.broadcasted_iota(jnp.int32, (16, 16 * D), 1)
    bd = jnp.where(r_iota == c_iota // D, 1.0, 0.0)
    cmat = lax.dot_general(a, bd, (((0,), (0,)), ((), ())),
                           preferred_element_type=jnp.float32)
    dmat = cmat - a.reshape(1, 16 * D)
    e = jnp.exp2(-(dmat * dmat))
    P = lax.dot_general(e, bd, (((1,), (1,)), ((), ())),
                        preferred_element_type=jnp.float32)
    tot = jnp.sum(P, axis=0, keepdims=True)
    PK = P / tot
    pk = PK[:, 0:8]
    qk = PK[:, 8:16]
    kl_elem = pk * (jnp.log(pk) - jnp.log(qk))
    kl = jnp.sum(jnp.sum(kl_elem, axis=1, keepdims=True), axis=0,
                 keepdims=True)
    out_ref[:, :] = 0.5 * kl


def kernel(pair_now, person_1_now, person_2_now, scene_now, pair_pre,
           person_1_pre, person_2_pre, scene_pre, weight, w_ih, w_hh,
           b_ih, b_hh):
    arrs = (pair_now, person_1_now, person_2_now, scene_now,
            pair_pre, person_1_pre, person_2_pre, scene_pre)
    parts = _get_sc_partial_sums()(*arrs)
    parts = parts.reshape(NA * NW, D)
    tc_sums = _tc_reduce(arrs)

    noise = jnp.asarray(_NOISE)

    out = pl.pallas_call(
        _tc_finish_body,
        out_shape=jax.ShapeDtypeStruct((1, 1), jnp.float32),
    )(parts, tc_sums, weight, w_ih, w_hh, b_ih.reshape(1, 3 * D),
      b_hh.reshape(1, 3 * D), noise)
    return out[0, 0]


# speedup vs baseline: 1.0081x; 1.0081x over previous
import functools

import jax
import jax.numpy as jnp
import numpy as np
from jax import lax
from jax.experimental import pallas as pl
from jax.experimental.pallas import tpu as pltpu
from jax.experimental.pallas import tpu_sc as plsc

T, D = 8192, 512
NUM_LAYERS = 3
NA = 8
NW = 32
T_SC = 1024
T_TC = T - T_SC
RPW = T_SC // NW
CH = min(64, RPW)
NCH = RPW // CH
NV = D // 16
TCB = 512
KDE_FACTOR = (D * 3.0 / 4.0) ** (-0.2)


def _noise_const():
    base_key = jax.random.key(1234)
    rows = [np.asarray(jax.random.normal(jax.random.fold_in(base_key, i),
                                         (D,), dtype=jnp.float32))
            for i in range(16)]
    order = list(range(0, 16, 2)) + list(range(1, 16, 2))
    return np.stack([rows[i] for i in order])


_NOISE = _noise_const()

@functools.lru_cache(maxsize=None)
def _get_sc_partial_sums():
    mesh = plsc.VectorSubcoreMesh(core_axis_name="c", subcore_axis_name="s")

    @functools.partial(
        pl.kernel,
        out_type=jax.ShapeDtypeStruct((NA, NW, D), jnp.float32),
        mesh=mesh,
        scratch_types=[
            pltpu.VMEM((CH, D), jnp.float32),
            pltpu.VMEM((CH, D), jnp.float32),
            pltpu.VMEM((NA, D), jnp.float32),
            pltpu.SemaphoreType.DMA,
            pltpu.SemaphoreType.DMA,
            pltpu.SemaphoreType.DMA,
        ],
    )
    def _sc_partial_sums(a0, a1, a2, a3, a4, a5, a6, a7, out,
                         buf0, buf1, accv, sem0, sem1, osem):
        wid = lax.axis_index("s") * 2 + lax.axis_index("c")
        base = wid * RPW
        refs = (a0, a1, a2, a3, a4, a5, a6, a7)
        bufs = (buf0, buf1)
        sems = (sem0, sem1)
        sched = [(a, c) for a in range(NA) for c in range(NCH)]

        def issue(k):
            a, c = sched[k]
            return pltpu.async_copy(
                refs[a].at[pl.ds(base + c * CH, CH)], bufs[k % 2],
                sems[k % 2])

        copies = [None, None]
        copies[0] = issue(0)
        out_copies = []
        accs = None
        for k, (a, c) in enumerate(sched):
            if k + 1 < len(sched):
                copies[(k + 1) % 2] = issue(k + 1)
            copies[k % 2].wait()
            buf = bufs[k % 2]
            if c == 0:
                accs = tuple(jnp.zeros((16,), jnp.float32)
                             for _ in range(NV))

            def row_body(r, acc, buf=buf):
                return tuple(acc[j] + buf[r, pl.ds(16 * j, 16)]
                             for j in range(NV))

            accs = plsc.parallel_loop(0, CH, 1, unroll=2,
                                      carry=accs)(row_body)
            if c == NCH - 1:
                for j in range(NV):
                    accv[a, pl.ds(16 * j, 16)] = accs[j]
                out_copies.append(
                    pltpu.async_copy(accv.at[a], out.at[a, wid], osem))
        for oc in out_copies:
            oc.wait()

    return _sc_partial_sums


def _tc_reduce_body(a0, a1, a2, a3, a4, a5, a6, a7, out_ref):
    @pl.when(pl.program_id(0) == 0)
    def _():
        out_ref[...] = jnp.zeros_like(out_ref)

    for i, ref in enumerate((a0, a1, a2, a3, a4, a5, a6, a7)):
        out_ref[i:i + 1, :] += jnp.sum(ref[...], axis=0, keepdims=True)


def _tc_reduce(arrs):
    grid = (T_TC // TCB,)
    in_spec = pl.BlockSpec((TCB, D), lambda i: (i + T_SC // TCB, 0))
    return pl.pallas_call(
        _tc_reduce_body,
        grid=grid,
        in_specs=[in_spec] * NA,
        out_specs=pl.BlockSpec((NA, D), lambda i: (0, 0)),
        out_shape=jax.ShapeDtypeStruct((NA, D), jnp.float32),
    )(*arrs)


def _split3(g):
    return g[:, :D], g[:, D:2 * D], g[:, 2 * D:]


def _tc_finish_body(parts_ref, tcsum_ref, w_ref, wih_ref, whh_ref, bih_ref,
                    bhh_ref, noise_ref, out_ref):
    sums = [jnp.sum(parts_ref[a * NW:(a + 1) * NW, :], axis=0, keepdims=True)
            for a in range(NA)]
    S = jnp.concatenate(sums, axis=0) + tcsum_ref[...]
    means = S * (1.0 / T)

    H0 = means[0:4]
    hp = means[4:8]
    bi = bih_ref[...]
    bh = bhh_ref[...]
    dn_t = (((1,), (1,)), ((), ()))

    for l in range(NUM_LAYERS):
        s_l = jnp.sum(hp, axis=0, keepdims=True)
        agg0 = lax.dot_general(s_l, w_ref[l], (((1,), (0,)), ((), ())),
                               preferred_element_type=jnp.float32)
        gi0 = lax.dot_general(agg0, wih_ref[...], dn_t,
                              preferred_element_type=jnp.float32) + bi
        gh0 = lax.dot_general(H0, whh_ref[...], dn_t,
                              preferred_element_type=jnp.float32) + bh
        ghp = lax.dot_general(hp, whh_ref[...], dn_t,
                              preferred_element_type=jnp.float32) + bh
        i_r, i_z, i_n = _split3(gi0)
        h_r, h_z, h_n = _split3(gh0)
        r = jax.nn.sigmoid(i_r + h_r)
        z = jax.nn.sigmoid(i_z + h_z)
        n = jnp.tanh(i_n + r * h_n)
        H0 = (1.0 - z) * n + z * H0
        i_rp, i_zp, i_np = _split3(bi)
        h_rp, h_zp, h_np = _split3(ghp)
        rp = jax.nn.sigmoid(i_rp + h_rp)
        zp = jax.nn.sigmoid(i_zp + h_zp)
        np_ = jnp.tanh(i_np + rp * h_np)
        hp = (1.0 - zp) * np_ + zp * hp

    ones_row = jnp.ones((1, D), jnp.float32)
    ones_col = jnp.ones((D, 1), jnp.float32)
    c_fold = float(np.sqrt(np.log2(np.e) * 0.5))

    def pk_of(s):
        msum = jnp.sum(s, axis=1, keepdims=True) * (1.0 / D)
        var = jnp.sum((s - msum) ** 2, axis=1, keepdims=True) * (1.0 / (D - 1))
        a = s * (c_fold / (jnp.sqrt(var) * KDE_FACTOR))
        col = lax.dot_general(a, ones_row, (((0,), (0,)), ((), ())),
                              preferred_element_type=jnp.float32)
        d = col - a
        e = jnp.exp2(-(d * d))
        p = lax.dot_general(e, ones_col, (((1,), (0,)), ((), ())),
                            preferred_element_type=jnp.float32)
        total = jnp.sum(p, axis=0, keepdims=True)
        return p / total

    acc = jnp.zeros((1, 1), jnp.float32)
    for i in range(8):
        t1 = H0[i % 4:i % 4 + 1]
        t2 = S[4 + i % 4:5 + i % 4] if i < 4 else S[i % 4:i % 4 + 1]
        s1 = t1 + 0.01 * noise_ref[i:i + 1, :]
        s2 = t2 + 0.01 * noise_ref[8 + i:9 + i, :]
        pk = pk_of(s1)
        qk = pk_of(s2)
        kl = jnp.sum(pk * jnp.log(pk / qk), axis=0, keepdims=True)
        acc = acc + kl
    out_ref[:, :] = 0.5 * acc


def kernel(pair_now, person_1_now, person_2_now, scene_now, pair_pre,
           person_1_pre, person_2_pre, scene_pre, weight, w_ih, w_hh,
           b_ih, b_hh):
    arrs = (pair_now, person_1_now, person_2_now, scene_now,
            pair_pre, person_1_pre, person_2_pre, scene_pre)
    parts = _get_sc_partial_sums()(*arrs)
    parts = parts.reshape(NA * NW, D)
    tc_sums = _tc_reduce(arrs)

    noise = jnp.asarray(_NOISE)

    out = pl.pallas_call(
        _tc_finish_body,
        out_shape=jax.ShapeDtypeStruct((1, 1), jnp.float32),
    )(parts, tc_sums, weight, w_ih, w_hh, b_ih.reshape(1, 3 * D),
      b_hh.reshape(1, 3 * D), noise)
    return out[0, 0]


# speedup vs baseline: 1.0108x; 1.0027x over previous
import functools

import jax
import jax.numpy as jnp
import numpy as np
from jax import lax
from jax.experimental import pallas as pl
from jax.experimental.pallas import tpu as pltpu
from jax.experimental.pallas import tpu_sc as plsc

T, D = 8192, 512
NUM_LAYERS = 3
NA = 8
NW = 32
T_SC = 512
T_TC = T - T_SC
RPW = T_SC // NW
CH = min(64, RPW)
NCH = RPW // CH
NV = D // 16
TCB = 512
KDE_FACTOR = (D * 3.0 / 4.0) ** (-0.2)


def _noise_const():
    base_key = jax.random.key(1234)
    rows = [np.asarray(jax.random.normal(jax.random.fold_in(base_key, i),
                                         (D,), dtype=jnp.float32))
            for i in range(16)]
    order = list(range(0, 16, 2)) + list(range(1, 16, 2))
    return np.stack([rows[i] for i in order])


_NOISE = _noise_const()

@functools.lru_cache(maxsize=None)
def _get_sc_partial_sums():
    mesh = plsc.VectorSubcoreMesh(core_axis_name="c", subcore_axis_name="s")

    @functools.partial(
        pl.kernel,
        out_type=jax.ShapeDtypeStruct((NA, NW, D), jnp.float32),
        mesh=mesh,
        scratch_types=[
            pltpu.VMEM((CH, D), jnp.float32),
            pltpu.VMEM((CH, D), jnp.float32),
            pltpu.VMEM((NA, D), jnp.float32),
            pltpu.SemaphoreType.DMA,
            pltpu.SemaphoreType.DMA,
            pltpu.SemaphoreType.DMA,
        ],
    )
    def _sc_partial_sums(a0, a1, a2, a3, a4, a5, a6, a7, out,
                         buf0, buf1, accv, sem0, sem1, osem):
        wid = lax.axis_index("s") * 2 + lax.axis_index("c")
        base = wid * RPW
        refs = (a0, a1, a2, a3, a4, a5, a6, a7)
        bufs = (buf0, buf1)
        sems = (sem0, sem1)
        sched = [(a, c) for a in range(NA) for c in range(NCH)]

        def issue(k):
            a, c = sched[k]
            return pltpu.async_copy(
                refs[a].at[pl.ds(base + c * CH, CH)], bufs[k % 2],
                sems[k % 2])

        copies = [None, None]
        copies[0] = issue(0)
        out_copies = []
        accs = None
        for k, (a, c) in enumerate(sched):
            if k + 1 < len(sched):
                copies[(k + 1) % 2] = issue(k + 1)
            copies[k % 2].wait()
            buf = bufs[k % 2]
            if c == 0:
                accs = tuple(jnp.zeros((16,), jnp.float32)
                             for _ in range(NV))

            def row_body(r, acc, buf=buf):
                return tuple(acc[j] + buf[r, pl.ds(16 * j, 16)]
                             for j in range(NV))

            accs = plsc.parallel_loop(0, CH, 1, unroll=2,
                                      carry=accs)(row_body)
            if c == NCH - 1:
                for j in range(NV):
                    accv[a, pl.ds(16 * j, 16)] = accs[j]
                out_copies.append(
                    pltpu.async_copy(accv.at[a], out.at[a, wid], osem))
        for oc in out_copies:
            oc.wait()

    return _sc_partial_sums


def _tc_reduce_body(a0, a1, a2, a3, a4, a5, a6, a7, out_ref):
    @pl.when(pl.program_id(0) == 0)
    def _():
        out_ref[...] = jnp.zeros_like(out_ref)

    for i, ref in enumerate((a0, a1, a2, a3, a4, a5, a6, a7)):
        out_ref[i:i + 1, :] += jnp.sum(ref[...], axis=0, keepdims=True)


def _tc_reduce(arrs):
    grid = (T_TC // TCB,)
    in_spec = pl.BlockSpec((TCB, D), lambda i: (i + T_SC // TCB, 0))
    return pl.pallas_call(
        _tc_reduce_body,
        grid=grid,
        in_specs=[in_spec] * NA,
        out_specs=pl.BlockSpec((NA, D), lambda i: (0, 0)),
        out_shape=jax.ShapeDtypeStruct((NA, D), jnp.float32),
    )(*arrs)


def _split3(g):
    return g[:, :D], g[:, D:2 * D], g[:, 2 * D:]


def _tc_finish_body(parts_ref, tcsum_ref, w_ref, wih_ref, whh_ref, bih_ref,
                    bhh_ref, noise_ref, out_ref):
    sums = [jnp.sum(parts_ref[a * NW:(a + 1) * NW, :], axis=0, keepdims=True)
            for a in range(NA)]
    S = jnp.concatenate(sums, axis=0) + tcsum_ref[...]
    means = S * (1.0 / T)

    H0 = means[0:4]
    hp = means[4:8]
    bi = bih_ref[...]
    bh = bhh_ref[...]
    dn_t = (((1,), (1,)), ((), ()))

    for l in range(NUM_LAYERS):
        s_l = jnp.sum(hp, axis=0, keepdims=True)
        agg0 = lax.dot_general(s_l, w_ref[l], (((1,), (0,)), ((), ())),
                               preferred_element_type=jnp.float32)
        gi0 = lax.dot_general(agg0, wih_ref[...], dn_t,
                              preferred_element_type=jnp.float32) + bi
        gh0 = lax.dot_general(H0, whh_ref[...], dn_t,
                              preferred_element_type=jnp.float32) + bh
        ghp = lax.dot_general(hp, whh_ref[...], dn_t,
                              preferred_element_type=jnp.float32) + bh
        i_r, i_z, i_n = _split3(gi0)
        h_r, h_z, h_n = _split3(gh0)
        r = jax.nn.sigmoid(i_r + h_r)
        z = jax.nn.sigmoid(i_z + h_z)
        n = jnp.tanh(i_n + r * h_n)
        H0 = (1.0 - z) * n + z * H0
        i_rp, i_zp, i_np = _split3(bi)
        h_rp, h_zp, h_np = _split3(ghp)
        rp = jax.nn.sigmoid(i_rp + h_rp)
        zp = jax.nn.sigmoid(i_zp + h_zp)
        np_ = jnp.tanh(i_np + rp * h_np)
        hp = (1.0 - zp) * np_ + zp * hp

    ones_row = jnp.ones((1, D), jnp.float32)
    ones_col = jnp.ones((D, 1), jnp.float32)
    c_fold = float(np.sqrt(np.log2(np.e) * 0.5))

    def pk_of(s):
        msum = jnp.sum(s, axis=1, keepdims=True) * (1.0 / D)
        var = jnp.sum((s - msum) ** 2, axis=1, keepdims=True) * (1.0 / (D - 1))
        a = s * (c_fold / (jnp.sqrt(var) * KDE_FACTOR))
        col = lax.dot_general(a, ones_row, (((0,), (0,)), ((), ())),
                              preferred_element_type=jnp.float32)
        d = col - a
        e = jnp.exp2(-(d * d))
        p = lax.dot_general(e, ones_col, (((1,), (0,)), ((), ())),
                            preferred_element_type=jnp.float32)
        total = jnp.sum(p, axis=0, keepdims=True)
        return p / total

    acc = jnp.zeros((1, 1), jnp.float32)
    for i in range(8):
        t1 = H0[i % 4:i % 4 + 1]
        t2 = S[4 + i % 4:5 + i % 4] if i < 4 else S[i % 4:i % 4 + 1]
        s1 = t1 + 0.01 * noise_ref[i:i + 1, :]
        s2 = t2 + 0.01 * noise_ref[8 + i:9 + i, :]
        pk = pk_of(s1)
        qk = pk_of(s2)
        kl = jnp.sum(pk * jnp.log(pk / qk), axis=0, keepdims=True)
        acc = acc + kl
    out_ref[:, :] = 0.5 * acc


def kernel(pair_now, person_1_now, person_2_now, scene_now, pair_pre,
           person_1_pre, person_2_pre, scene_pre, weight, w_ih, w_hh,
           b_ih, b_hh):
    arrs = (pair_now, person_1_now, person_2_now, scene_now,
            pair_pre, person_1_pre, person_2_pre, scene_pre)
    parts = _get_sc_partial_sums()(*arrs)
    parts = parts.reshape(NA * NW, D)
    tc_sums = _tc_reduce(arrs)

    noise = jnp.asarray(_NOISE)

    out = pl.pallas_call(
        _tc_finish_body,
        out_shape=jax.ShapeDtypeStruct((1, 1), jnp.float32),
    )(parts, tc_sums, weight, w_ih, w_hh, b_ih.reshape(1, 3 * D),
      b_hh.reshape(1, 3 * D), noise)
    return out[0, 0]
